# Initial kernel scaffold; baseline (speedup 1.0000x reference)
import jax
import jax.numpy as jnp
from jax.experimental import pallas as pl


def kernel(text, emb_weight, fc_weight):
    raise NotImplementedError("write your pallas kernel here")


# speedup vs baseline: 33.1226x; 33.1226x over previous
import functools

import jax
import jax.numpy as jnp
from jax import lax
from jax.experimental import pallas as pl
from jax.experimental.pallas import tpu as pltpu
from jax.experimental.pallas import tpu_sc as plsc

_VOCAB = 100000
_D = 128
_B = 4096
_L = 200
_OFFSET = 0.5

_ROWS_PER_BLK = 4000
_NBLK = _VOCAB // _ROWS_PER_BLK


def _matvec_body(emb_ref, fc_ref, out_ref):
    e = jnp.clip(emb_ref[...], 0.0, 1.0)
    w = jnp.clip(fc_ref[...], 0.0, 1.0)
    r = lax.dot_general(w, e, (((1,), (1,)), ((), ())),
                        preferred_element_type=jnp.float32)
    out_ref[...] = r.reshape(1, 1, _ROWS_PER_BLK)


def _compute_s(emb_weight, fc_weight):
    s3 = pl.pallas_call(
        _matvec_body,
        grid=(_NBLK,),
        in_specs=[
            pl.BlockSpec((_ROWS_PER_BLK, _D), lambda i: (i, 0)),
            pl.BlockSpec((1, _D), lambda i: (0, 0)),
        ],
        out_specs=pl.BlockSpec((1, 1, _ROWS_PER_BLK), lambda i: (i, 0, 0)),
        out_shape=jax.ShapeDtypeStruct((_NBLK, 1, _ROWS_PER_BLK), jnp.float32),
    )(emb_weight, fc_weight)
    return s3.reshape(_VOCAB)


_NW = 32
_BPW = _B // _NW
_NCHUNK = _BPW // 16


_IPW = _L * _BPW


def _sc_body(text_r, s_vec, out, idx_v, vals_v, out_v, sem):
    wid = lax.axis_index("s") * 2 + lax.axis_index("c")
    base = wid * _BPW
    pltpu.sync_copy(text_r.at[wid], idx_v)
    pltpu.async_copy(s_vec.at[idx_v], vals_v, sem).wait()

    def body(l, accs):
        return tuple(acc + vals_v[pl.ds(l * _BPW + c * 16, 16)]
                     for c, acc in enumerate(accs))

    accs = lax.fori_loop(
        0, _L, body,
        tuple(jnp.zeros((16,), jnp.float32) for _ in range(_NCHUNK)))
    for c in range(_NCHUNK):
        out_v[pl.ds(c * 16, 16)] = accs[c] * (1.0 / _L) - _OFFSET
    pltpu.sync_copy(out_v, out.at[pl.ds(base, _BPW)])


@functools.cache
def _sc_pool():
    return functools.partial(
        pl.kernel,
        mesh=plsc.VectorSubcoreMesh(core_axis_name="c", subcore_axis_name="s"),
        out_type=jax.ShapeDtypeStruct((_B,), jnp.float32),
        scratch_types=[
            pltpu.VMEM((_IPW,), jnp.int32),
            pltpu.VMEM((_IPW,), jnp.float32),
            pltpu.VMEM((_BPW,), jnp.float32),
            pltpu.SemaphoreType.DMA,
        ],
    )(_sc_body)


def kernel(text, emb_weight, fc_weight):
    s = _compute_s(emb_weight, fc_weight)
    text_r = (text.astype(jnp.int32)
              .reshape(_NW, _BPW, _L)
              .transpose(0, 2, 1)
              .reshape(_NW, _IPW))
    return _sc_pool()(text_r, s)


# speedup vs baseline: 44.8849x; 1.3551x over previous
import functools

import jax
import jax.numpy as jnp
from jax import lax
from jax.experimental import pallas as pl
from jax.experimental.pallas import tpu as pltpu
from jax.experimental.pallas import tpu_sc as plsc

_VOCAB = 100000
_D = 128
_B = 4096
_L = 200
_OFFSET = 0.5

_ROWS_PER_BLK = 4000
_NBLK = _VOCAB // _ROWS_PER_BLK


def _matvec_body(emb_ref, fc_ref, out_ref):
    e = jnp.clip(emb_ref[...], 0.0, 1.0)
    w = jnp.clip(fc_ref[...], 0.0, 1.0)
    r = lax.dot_general(w, e, (((1,), (1,)), ((), ())),
                        preferred_element_type=jnp.float32)
    out_ref[...] = r.reshape(1, 1, _ROWS_PER_BLK)


def _compute_s(emb_weight, fc_weight):
    s3 = pl.pallas_call(
        _matvec_body,
        grid=(_NBLK,),
        in_specs=[
            pl.BlockSpec((_ROWS_PER_BLK, _D), lambda i: (i, 0)),
            pl.BlockSpec((1, _D), lambda i: (0, 0)),
        ],
        out_specs=pl.BlockSpec((1, 1, _ROWS_PER_BLK), lambda i: (i, 0, 0)),
        out_shape=jax.ShapeDtypeStruct((_NBLK, 1, _ROWS_PER_BLK), jnp.float32),
    )(emb_weight, fc_weight)
    return s3.reshape(_VOCAB)


_NW = 32
_BPW = _B // _NW
_NCHUNK = _BPW // 16


_IPW = _L * _BPW
_VPAD = 100096


def _sc_body(text_r, s_vec, out, idx_v, s_t, out_v, sem, sem2):
    wid = lax.axis_index("s") * 2 + lax.axis_index("c")
    base = wid * _BPW
    cp1 = pltpu.make_async_copy(text_r.at[wid], idx_v, sem)
    cp2 = pltpu.make_async_copy(s_vec, s_t, sem2)
    cp1.start()
    cp2.start()
    cp1.wait()
    cp2.wait()

    def body(l, accs):
        new = []
        for c in range(_NCHUNK):
            iv = idx_v[pl.ds(l * _BPW + c * 16, 16)]
            vals = plsc.load_gather(s_t, [iv])
            new.append(accs[c] + vals)
        return tuple(new)

    accs = lax.fori_loop(
        0, _L, body,
        tuple(jnp.zeros((16,), jnp.float32) for _ in range(_NCHUNK)))
    for c in range(_NCHUNK):
        out_v[pl.ds(c * 16, 16)] = accs[c] * (1.0 / _L) - _OFFSET
    pltpu.sync_copy(out_v, out.at[pl.ds(base, _BPW)])


@functools.cache
def _sc_pool():
    return functools.partial(
        pl.kernel,
        mesh=plsc.VectorSubcoreMesh(core_axis_name="c", subcore_axis_name="s"),
        compiler_params=pltpu.CompilerParams(needs_layout_passes=False),
        out_type=jax.ShapeDtypeStruct((_B,), jnp.float32),
        scratch_types=[
            pltpu.VMEM((_IPW,), jnp.int32),
            pltpu.VMEM((_VPAD,), jnp.float32),
            pltpu.VMEM((_BPW,), jnp.float32),
            pltpu.SemaphoreType.DMA,
            pltpu.SemaphoreType.DMA,
        ],
    )(_sc_body)


def kernel(text, emb_weight, fc_weight):
    s = jnp.pad(_compute_s(emb_weight, fc_weight), (0, _VPAD - _VOCAB))
    text_r = (text.astype(jnp.int32)
              .reshape(_NW, _BPW, _L)
              .transpose(0, 2, 1)
              .reshape(_NW, _IPW))
    return _sc_pool()(text_r, s)
